# baseline (speedup 1.0000x reference)
import jax
import jax.numpy as jnp
from jax.experimental import pallas as pl
from jax.experimental.pallas import tpu as pltpu

_K = 1024
_C = 256
_B = 8
_P = 1024
_BETA = 0.25
_EPS = 1e-12
_ENT_EPS = 1e-4


def _vq_body(z_ref, ew_ref, zq_ref, idx_ref, loss_ref, ent_ref,
             ewn_ref, ewthi_ref, ewtlo_ref, usage_ref):
    b = pl.program_id(0)
    nb = pl.num_programs(0)

    @pl.when(b == 0)
    def _init():
        ew = ew_ref[...]
        norm = jnp.sqrt(jnp.sum(ew * ew, axis=1, keepdims=True))
        ewn = ew / jnp.maximum(norm, _EPS)
        ewn_ref[...] = ewn
        ewt = ewn.T
        hi = ewt.astype(jnp.bfloat16)
        ewthi_ref[...] = hi
        ewtlo_ref[...] = (ewt - hi.astype(jnp.float32)).astype(jnp.bfloat16)
        usage_ref[...] = jnp.zeros_like(usage_ref)
        loss_ref[...] = jnp.zeros_like(loss_ref)

    cdims = (((1,), (0,)), ((), ()))
    z = z_ref[0].reshape(_C, _P)
    s2 = jnp.sum(z * z, axis=0, keepdims=True)
    zn = z / jnp.maximum(jnp.sqrt(s2), _EPS)
    sim = jax.lax.dot_general(
        ewn_ref[...].astype(jnp.bfloat16), zn.astype(jnp.bfloat16), cdims,
        preferred_element_type=jnp.float32)
    smax = jnp.max(sim, axis=0, keepdims=True)
    kiota = jax.lax.broadcasted_iota(jnp.int32, sim.shape, 0)
    idx = jnp.min(jnp.where(sim == smax, kiota, jnp.int32(2**30)),
                  axis=0, keepdims=True)
    idx_ref[pl.ds(b, 1), :] = idx

    onehot = (kiota == idx).astype(jnp.float32)
    usage_ref[...] += jnp.sum(onehot, axis=1, keepdims=True)
    oh16 = onehot.astype(jnp.bfloat16)
    zq = (jax.lax.dot_general(ewthi_ref[...], oh16, cdims,
                              preferred_element_type=jnp.float32)
          + jax.lax.dot_general(ewtlo_ref[...], oh16, cdims,
                                preferred_element_type=jnp.float32))
    zq_ref[0] = zq.reshape(_C, 32, 32)
    diff = zq - zn
    loss_ref[...] += jnp.sum(diff * diff).reshape(1, 1)

    @pl.when(b == nb - 1)
    def _finish():
        total = jnp.float32(_B * _P)
        loss_ref[...] = (1.0 + _BETA) * (loss_ref[...] / total)
        pe = usage_ref[...] + _ENT_EPS
        probs = pe / jnp.sum(pe)
        ent_ref[...] = -jnp.sum(probs * jnp.log(probs)).reshape(1, 1)


def kernel(z, embedding_weight):
    zq, idx, loss, ent = pl.pallas_call(
        _vq_body,
        grid=(_B,),
        in_specs=[
            pl.BlockSpec((1, _C, 32, 32), lambda b: (b, 0, 0, 0)),
            pl.BlockSpec((_K, _C), lambda b: (0, 0)),
        ],
        out_specs=[
            pl.BlockSpec((1, _C, 32, 32), lambda b: (b, 0, 0, 0)),
            pl.BlockSpec((_B, _P), lambda b: (0, 0)),
            pl.BlockSpec((1, 1), lambda b: (0, 0)),
            pl.BlockSpec((1, 1), lambda b: (0, 0)),
        ],
        out_shape=[
            jax.ShapeDtypeStruct((_B, _C, 32, 32), jnp.float32),
            jax.ShapeDtypeStruct((_B, _P), jnp.int32),
            jax.ShapeDtypeStruct((1, 1), jnp.float32),
            jax.ShapeDtypeStruct((1, 1), jnp.float32),
        ],
        scratch_shapes=[
            pltpu.VMEM((_K, _C), jnp.float32),
            pltpu.VMEM((_C, _K), jnp.bfloat16),
            pltpu.VMEM((_C, _K), jnp.bfloat16),
            pltpu.VMEM((_K, 1), jnp.float32),
        ],
    )(z, embedding_weight)
    return (zq, loss[0, 0], ent[0, 0], idx)


# speedup vs baseline: 1.3481x; 1.3481x over previous
import jax
import jax.numpy as jnp
from jax.experimental import pallas as pl
from jax.experimental.pallas import tpu as pltpu

_K = 1024
_C = 256
_B = 8
_P = 1024
_BETA = 0.25
_EPS = 1e-12
_ENT_EPS = 1e-4


def _vq_body(z_ref, ew_ref, zq_ref, idx_ref, loss_ref, ent_ref,
             ewn_ref, ewthi_ref, ewtlo_ref, usage_ref):
    b = pl.program_id(0)
    nb = pl.num_programs(0)

    @pl.when(b == 0)
    def _init():
        ew = ew_ref[...]
        norm = jnp.sqrt(jnp.sum(ew * ew, axis=1, keepdims=True))
        ewn = ew / jnp.maximum(norm, _EPS)
        ewn_ref[...] = ewn
        ewt = ewn.T
        hi = ewt.astype(jnp.bfloat16)
        ewthi_ref[...] = hi
        ewtlo_ref[...] = (ewt - hi.astype(jnp.float32)).astype(jnp.bfloat16)
        usage_ref[...] = jnp.zeros_like(usage_ref)
        loss_ref[...] = jnp.zeros_like(loss_ref)

    cdims = (((1,), (0,)), ((), ()))
    z = z_ref[0]
    s2 = jnp.sum(z * z, axis=0, keepdims=True)
    zn = z / jnp.maximum(jnp.sqrt(s2), _EPS)
    sim = jax.lax.dot_general(
        ewn_ref[...].astype(jnp.bfloat16), zn.astype(jnp.bfloat16), cdims,
        preferred_element_type=jnp.float32)
    smax = jnp.max(sim, axis=0, keepdims=True)
    kiota = jax.lax.broadcasted_iota(jnp.int32, sim.shape, 0)
    idx = jnp.min(jnp.where(sim == smax, kiota, jnp.int32(2**30)),
                  axis=0, keepdims=True)
    idx_ref[pl.ds(b, 1), :] = idx

    onehot = (kiota == idx).astype(jnp.float32)
    usage_ref[...] += jnp.sum(onehot, axis=1, keepdims=True)
    oh16 = onehot.astype(jnp.bfloat16)
    zq = (jax.lax.dot_general(ewthi_ref[...], oh16, cdims,
                              preferred_element_type=jnp.float32)
          + jax.lax.dot_general(ewtlo_ref[...], oh16, cdims,
                                preferred_element_type=jnp.float32))
    zq_ref[0] = zq.reshape(_C, 32, 32)
    diff = zq - zn
    loss_ref[...] += jnp.sum(diff * diff).reshape(1, 1)

    @pl.when(b == nb - 1)
    def _finish():
        total = jnp.float32(_B * _P)
        loss_ref[...] = (1.0 + _BETA) * (loss_ref[...] / total)
        pe = usage_ref[...] + _ENT_EPS
        probs = pe / jnp.sum(pe)
        ent_ref[...] = -jnp.sum(probs * jnp.log(probs)).reshape(1, 1)


def kernel(z, embedding_weight):
    zr = z.reshape(_B, _C, _P)
    zq, idx, loss, ent = pl.pallas_call(
        _vq_body,
        grid=(_B,),
        in_specs=[
            pl.BlockSpec((1, _C, _P), lambda b: (b, 0, 0)),
            pl.BlockSpec((_K, _C), lambda b: (0, 0)),
        ],
        out_specs=[
            pl.BlockSpec((1, _C, 32, 32), lambda b: (b, 0, 0, 0)),
            pl.BlockSpec((_B, _P), lambda b: (0, 0)),
            pl.BlockSpec((1, 1), lambda b: (0, 0)),
            pl.BlockSpec((1, 1), lambda b: (0, 0)),
        ],
        out_shape=[
            jax.ShapeDtypeStruct((_B, _C, 32, 32), jnp.float32),
            jax.ShapeDtypeStruct((_B, _P), jnp.int32),
            jax.ShapeDtypeStruct((1, 1), jnp.float32),
            jax.ShapeDtypeStruct((1, 1), jnp.float32),
        ],
        scratch_shapes=[
            pltpu.VMEM((_K, _C), jnp.float32),
            pltpu.VMEM((_C, _K), jnp.bfloat16),
            pltpu.VMEM((_C, _K), jnp.bfloat16),
            pltpu.VMEM((_K, 1), jnp.float32),
        ],
    )(zr, embedding_weight)
    return (zq, loss[0, 0], ent[0, 0], idx)
